# baseline (speedup 1.0000x reference)
import functools

import jax
import jax.numpy as jnp
from jax import lax
from jax.experimental import pallas as pl
from jax.experimental.pallas import tpu as pltpu
from jax.experimental.pallas import tpu_sc as plsc

D = 64
B = 1024
S = 200
TOTAL = B * S
NC, NS = 2, 16
NW = NC * NS
PER_W = TOTAL // NW
GATHER = 128
NG = PER_W // GATHER
LANES = 16
KD = D // LANES

_mesh = plsc.VectorSubcoreMesh(core_axis_name="c", subcore_axis_name="s")


@functools.partial(
    pl.kernel,
    out_type=jax.ShapeDtypeStruct((TOTAL, D), jnp.float32),
    mesh=_mesh,
    scratch_types=[
        pltpu.VMEM((NG, GATHER), jnp.int32),
        pltpu.VMEM((GATHER, D), jnp.float32),
        pltpu.VMEM((S, D), jnp.float32),
        pltpu.SemaphoreType.DMA,
    ],
    compiler_params=pltpu.CompilerParams(use_tc_tiling_on_sc=False),
)
def _emb_lookup(ids_hbm, table_hbm, pos_hbm, out_hbm, idx_v, gbuf, pos_v, sem):
    wid = lax.axis_index("s") * NC + lax.axis_index("c")
    pltpu.sync_copy(pos_hbm.at[pl.ds(0, S)], pos_v)
    pltpu.sync_copy(ids_hbm.at[wid], idx_v)

    def chunk(j, carry):
        pltpu.async_copy(table_hbm.at[idx_v.at[j]], gbuf, sem).wait()
        s0 = lax.rem(j * GATHER, S)

        def row(r, carry2):
            s = s0 + r
            s = jnp.where(s >= S, s - S, s)
            for k in range(KD):
                sl = pl.ds(k * LANES, LANES)
                gbuf[r, sl] = gbuf[r, sl] + pos_v[s, sl]
            return carry2

        lax.fori_loop(0, GATHER, row, 0)
        base = wid * PER_W + j * GATHER
        pltpu.sync_copy(gbuf, out_hbm.at[pl.ds(base, GATHER)])
        return carry

    lax.fori_loop(0, NG, chunk, 0)


def kernel(token_ids, token_table, pos_table):
    ids = token_ids.reshape(NW, NG, GATHER).astype(jnp.int32)
    out = _emb_lookup(ids, token_table, pos_table)
    return out.reshape(B, S, D)


# speedup vs baseline: 1.1953x; 1.1953x over previous
import functools

import jax
import jax.numpy as jnp
from jax import lax
from jax.experimental import pallas as pl
from jax.experimental.pallas import tpu as pltpu
from jax.experimental.pallas import tpu_sc as plsc

D = 64
B = 1024
S = 200
TOTAL = B * S
NC, NS = 2, 16
NW = NC * NS
PER_W = TOTAL // NW
C = 2 * S
NCH = PER_W // C
NBUF = 3
GS = (128, 128, 128, 16)
GOFF = (0, 128, 256, 384)
LANES = 16
KD = D // LANES

_mesh = plsc.VectorSubcoreMesh(core_axis_name="c", subcore_axis_name="s")


@functools.partial(
    pl.kernel,
    out_type=jax.ShapeDtypeStruct((TOTAL, D), jnp.float32),
    mesh=_mesh,
    scratch_types=[
        pltpu.VMEM((PER_W,), jnp.int32),
        pltpu.VMEM((NBUF, C, D), jnp.float32),
        pltpu.VMEM((S, D), jnp.float32),
        pltpu.SemaphoreType.DMA((NBUF,)),
        pltpu.SemaphoreType.DMA((NBUF,)),
    ],
    compiler_params=pltpu.CompilerParams(use_tc_tiling_on_sc=False),
)
def _emb_lookup(ids_hbm, table_hbm, pos_hbm, out_hbm, idx_v, gbuf, pos_v,
                gsem, ssem):
    wid = lax.axis_index("s") * NC + lax.axis_index("c")
    pltpu.sync_copy(pos_hbm.at[pl.ds(0, S)], pos_v)
    pltpu.sync_copy(ids_hbm.at[wid], idx_v)
    row0 = wid * PER_W

    def fire_gathers(c, b):
        hs = []
        for g, off in zip(GS, GOFF):
            hs.append(pltpu.async_copy(
                table_hbm.at[idx_v.at[pl.ds(c * C + off, g)]],
                gbuf.at[b, pl.ds(off, g)],
                gsem.at[b]))
        return hs

    def add_pos(b):
        @plsc.parallel_loop(0, S, unroll=4)
        def _(s):
            for k in range(KD):
                sl = pl.ds(k * LANES, LANES)
                p = pos_v[s, sl]
                gbuf[b, s, sl] = gbuf[b, s, sl] + p
                gbuf[b, S + s, sl] = gbuf[b, S + s, sl] + p

    gh = [None] * NBUF
    sh = [None] * NBUF
    gh[0] = fire_gathers(0, 0)
    for c in range(NCH):
        b = c % NBUF
        nb = (c + 1) % NBUF
        if c + 1 < NCH:
            if sh[nb] is not None:
                sh[nb].wait()
                sh[nb] = None
            gh[nb] = fire_gathers(c + 1, nb)
        for h in gh[b]:
            h.wait()
        add_pos(b)
        sh[b] = pltpu.async_copy(
            gbuf.at[b], out_hbm.at[pl.ds(row0 + c * C, C)], ssem.at[b])
    for h in sh:
        if h is not None:
            h.wait()


def kernel(token_ids, token_table, pos_table):
    ids = token_ids.reshape(NW, PER_W).astype(jnp.int32)
    out = _emb_lookup(ids, token_table, pos_table)
    return out.reshape(B, S, D)
